# baseline (speedup 1.0000x reference)
import functools

import jax
import jax.numpy as jnp
from jax import lax
from jax.experimental import pallas as pl
from jax.experimental.pallas import tpu as pltpu
from jax.experimental.pallas import tpu_sc as plsc

_VECTOR_DIM = 32
_ROWS_PER_BLOCK = 512


def _argmin_body(x_ref, c_ref, idx_ref, loss_ref, ct_ref):
    i = pl.program_id(0)
    x = x_ref[...]
    c = c_ref[...]
    n_cols = c.shape[1]
    sim2 = jnp.dot(x * -2.0, c, preferred_element_type=jnp.float32)
    rown = jnp.sum(x * x, axis=1, keepdims=True)
    coln = jnp.sum(c * c, axis=0, keepdims=True)
    dist = (rown + coln) + sim2
    m = jnp.min(dist, axis=1, keepdims=True)
    del n_cols
    idx_ref[0, 0, :] = jnp.argmin(dist, axis=1).astype(jnp.int32)

    @pl.when(i == 0)
    def _init():
        loss_ref[0, 0] = 0.0
        ct_ref[...] = c.T

    loss_ref[0, 0] += jnp.sum(m)


def _compute_indices_and_loss(x, codebook):
    n, _ = x.shape
    v = codebook.shape[1]
    r = _ROWS_PER_BLOCK
    g = n // r
    idx3, loss, ct = pl.pallas_call(
        _argmin_body,
        grid=(g,),
        in_specs=[
            pl.BlockSpec((r, _VECTOR_DIM), lambda i: (i, 0)),
            pl.BlockSpec((_VECTOR_DIM, v), lambda i: (0, 0)),
        ],
        out_specs=[
            pl.BlockSpec((1, 1, r), lambda i: (i, 0, 0)),
            pl.BlockSpec(memory_space=pltpu.SMEM),
            pl.BlockSpec((v, _VECTOR_DIM), lambda i: (0, 0)),
        ],
        out_shape=[
            jax.ShapeDtypeStruct((g, 1, r), jnp.int32),
            jax.ShapeDtypeStruct((1, 1), jnp.float32),
            jax.ShapeDtypeStruct((v, _VECTOR_DIM), jnp.float32),
        ],
    )(x, codebook)
    return idx3.reshape(n), loss[0, 0], ct


def _sc_gather(table, idx):
    n = idx.shape[0]
    d = table.shape[1]
    num_cores, num_subcores = 2, 16
    nw = num_cores * num_subcores
    b_per_w = n // nw

    mesh = plsc.VectorSubcoreMesh(core_axis_name="c", subcore_axis_name="s")

    @functools.partial(
        pl.kernel,
        mesh=mesh,
        compiler_params=pltpu.CompilerParams(use_tc_tiling_on_sc=False),
        out_type=jax.ShapeDtypeStruct((n, d), jnp.float32),
        scratch_types=[
            pltpu.VMEM((b_per_w,), jnp.int32),
            pltpu.VMEM((b_per_w, d), jnp.float32),
            pltpu.SemaphoreType.DMA,
        ],
    )
    def gather_kernel(table_hbm, idx_hbm, out_hbm, idx_v, rows_v, sem):
        wid = lax.axis_index("s") * num_cores + lax.axis_index("c")
        base = wid * b_per_w
        pltpu.sync_copy(idx_hbm.at[pl.ds(base, b_per_w)], idx_v)
        pltpu.async_copy(table_hbm.at[idx_v], rows_v, sem).wait()
        pltpu.sync_copy(rows_v, out_hbm.at[pl.ds(base, b_per_w)])

    return gather_kernel(table, idx)


def kernel(inputs, quantized_vectors):
    input_shape = inputs.shape
    x = inputs.reshape(-1, _VECTOR_DIM)
    n = x.shape[0]
    idx, loss_sum, ct = _compute_indices_and_loss(x, quantized_vectors)
    quantized = _sc_gather(ct, idx)
    vq_loss = loss_sum * (1.25 / (n * _VECTOR_DIM))
    return quantized.reshape(input_shape), vq_loss


# speedup vs baseline: 1.0520x; 1.0520x over previous
import functools

import jax
import jax.numpy as jnp
from jax import lax
from jax.experimental import pallas as pl
from jax.experimental.pallas import tpu as pltpu
from jax.experimental.pallas import tpu_sc as plsc

_VECTOR_DIM = 32
_ROWS_PER_BLOCK = 1024


def _argmin_body(x_ref, c_ref, idx_ref, loss_ref):
    i = pl.program_id(0)
    x = x_ref[...]
    c = c_ref[...]
    n_cols = c.shape[1]
    sim2 = jnp.dot(x * -2.0, c, preferred_element_type=jnp.float32)
    rown = jnp.sum(x * x, axis=1, keepdims=True)
    coln = jnp.sum(c * c, axis=0, keepdims=True)
    dist = (rown + coln) + sim2
    m = jnp.min(dist, axis=1, keepdims=True)
    del n_cols
    idx_ref[0, 0, :] = jnp.argmin(dist, axis=1).astype(jnp.int32)

    @pl.when(i == 0)
    def _init():
        loss_ref[0, 0] = 0.0

    loss_ref[0, 0] += jnp.sum(m)


def _compute_indices_and_loss(x, codebook):
    n, _ = x.shape
    v = codebook.shape[1]
    r = _ROWS_PER_BLOCK
    g = n // r
    idx3, loss = pl.pallas_call(
        _argmin_body,
        grid=(g,),
        in_specs=[
            pl.BlockSpec((r, _VECTOR_DIM), lambda i: (i, 0)),
            pl.BlockSpec((_VECTOR_DIM, v), lambda i: (0, 0)),
        ],
        out_specs=[
            pl.BlockSpec((1, 1, r), lambda i: (i, 0, 0)),
            pl.BlockSpec(memory_space=pltpu.SMEM),
        ],
        out_shape=[
            jax.ShapeDtypeStruct((g, 1, r), jnp.int32),
            jax.ShapeDtypeStruct((1, 1), jnp.float32),
        ],
    )(x, codebook)
    return idx3.reshape(n), loss[0, 0]


def _sc_gather(table, idx):
    n = idx.shape[0]
    d = table.shape[1]
    num_cores, num_subcores = 2, 16
    nw = num_cores * num_subcores
    b_per_w = n // nw

    mesh = plsc.VectorSubcoreMesh(core_axis_name="c", subcore_axis_name="s")

    @functools.partial(
        pl.kernel,
        mesh=mesh,
        compiler_params=pltpu.CompilerParams(use_tc_tiling_on_sc=False),
        out_type=jax.ShapeDtypeStruct((n, d), jnp.float32),
        scratch_types=[
            pltpu.VMEM((b_per_w,), jnp.int32),
            pltpu.VMEM((b_per_w, d), jnp.float32),
            pltpu.SemaphoreType.DMA,
        ],
    )
    def gather_kernel(table_hbm, idx_hbm, out_hbm, idx_v, rows_v, sem):
        wid = lax.axis_index("s") * num_cores + lax.axis_index("c")
        base = wid * b_per_w
        pltpu.sync_copy(idx_hbm.at[pl.ds(base, b_per_w)], idx_v)
        pltpu.async_copy(table_hbm.at[idx_v], rows_v, sem).wait()
        pltpu.sync_copy(rows_v, out_hbm.at[pl.ds(base, b_per_w)])

    return gather_kernel(table, idx)


def kernel(inputs, quantized_vectors):
    input_shape = inputs.shape
    x = inputs.reshape(-1, _VECTOR_DIM)
    n = x.shape[0]
    idx, loss_sum = _compute_indices_and_loss(x, quantized_vectors)
    quantized = _sc_gather(quantized_vectors.T, idx)
    vq_loss = loss_sum * (1.25 / (n * _VECTOR_DIM))
    return quantized.reshape(input_shape), vq_loss


# speedup vs baseline: 1.1728x; 1.1149x over previous
import functools

import jax
import jax.numpy as jnp
from jax import lax
from jax.experimental import pallas as pl
from jax.experimental.pallas import tpu as pltpu
from jax.experimental.pallas import tpu_sc as plsc

_VECTOR_DIM = 32
_ROWS_PER_BLOCK = 1024


def _argmin_body(x_ref, c_ref, idx_ref):
    i = pl.program_id(0)
    x = x_ref[...]
    c = c_ref[...]
    n_cols = c.shape[1]
    sim2 = jnp.dot(x * -2.0, c, preferred_element_type=jnp.float32)
    rown = jnp.sum(x * x, axis=1, keepdims=True)
    coln = jnp.sum(c * c, axis=0, keepdims=True)
    dist = (rown + coln) + sim2
    del i, n_cols
    idx_ref[0, 0, :] = jnp.argmin(dist, axis=1).astype(jnp.int32)


def _compute_indices_and_loss(x, codebook):
    n, _ = x.shape
    v = codebook.shape[1]
    r = _ROWS_PER_BLOCK
    g = n // r
    idx3 = pl.pallas_call(
        _argmin_body,
        grid=(g,),
        in_specs=[
            pl.BlockSpec((r, _VECTOR_DIM), lambda i: (i, 0)),
            pl.BlockSpec((_VECTOR_DIM, v), lambda i: (0, 0)),
        ],
        out_specs=pl.BlockSpec((1, 1, r), lambda i: (i, 0, 0)),
        out_shape=jax.ShapeDtypeStruct((g, 1, r), jnp.int32),
    )(x, codebook)
    return idx3.reshape(n)


def _sc_gather_and_loss(table, idx, x):
    n = idx.shape[0]
    d = table.shape[1]
    num_cores, num_subcores = 2, 16
    lanes = 16
    nw = num_cores * num_subcores
    b_per_w = n // nw

    mesh = plsc.VectorSubcoreMesh(core_axis_name="c", subcore_axis_name="s")

    @functools.partial(
        pl.kernel,
        mesh=mesh,
        compiler_params=pltpu.CompilerParams(use_tc_tiling_on_sc=False),
        out_type=[
            jax.ShapeDtypeStruct((n, d), jnp.float32),
            jax.ShapeDtypeStruct((nw, lanes), jnp.float32),
        ],
        scratch_types=[
            pltpu.VMEM((b_per_w,), jnp.int32),
            pltpu.VMEM((b_per_w, d), jnp.float32),
            pltpu.VMEM((b_per_w, d), jnp.float32),
            pltpu.VMEM((lanes,), jnp.float32),
            pltpu.SemaphoreType.DMA,
        ],
    )
    def gather_kernel(table_hbm, idx_hbm, x_hbm, out_hbm, part_hbm,
                      idx_v, rows_v, x_v, acc_v, sem):
        wid = lax.axis_index("s") * num_cores + lax.axis_index("c")
        base = wid * b_per_w
        pltpu.sync_copy(idx_hbm.at[pl.ds(base, b_per_w)], idx_v)
        cp_x = pltpu.async_copy(x_hbm.at[pl.ds(base, b_per_w)], x_v, sem)
        pltpu.async_copy(table_hbm.at[idx_v], rows_v, sem).wait()
        cp_x.wait()
        out_cp = pltpu.async_copy(rows_v, out_hbm.at[pl.ds(base, b_per_w)], sem)

        acc_v[...] = jnp.zeros((lanes,), jnp.float32)

        def body(r, _):
            a = acc_v[...]
            for h in range(d // lanes):
                dq = rows_v[r, pl.ds(h * lanes, lanes)] - x_v[r, pl.ds(h * lanes, lanes)]
                a = a + dq * dq
            acc_v[...] = a
            return _

        lax.fori_loop(0, b_per_w, body, 0, unroll=4)
        pltpu.sync_copy(acc_v, part_hbm.at[wid])
        out_cp.wait()

    return gather_kernel(table, idx, x)


def kernel(inputs, quantized_vectors):
    input_shape = inputs.shape
    x = inputs.reshape(-1, _VECTOR_DIM)
    n = x.shape[0]
    idx = _compute_indices_and_loss(x, quantized_vectors)
    quantized, partials = _sc_gather_and_loss(quantized_vectors.T, idx, x)
    vq_loss = jnp.sum(partials) * (1.25 / (n * _VECTOR_DIM))
    return quantized.reshape(input_shape), vq_loss
